# baseline (speedup 1.0000x reference)
import functools

import jax
import jax.numpy as jnp
from jax import lax
from jax.experimental import pallas as pl
from jax.experimental.pallas import tpu as pltpu
from jax.experimental.pallas import tpu_sc as plsc

BATCH = 16384
CTX_SIZE = 128

_info = plsc.get_sparse_core_info()
_NC, _NS = _info.num_cores, _info.num_subcores
_NW = _NC * _NS
_ROWS = BATCH // _NW

_mesh = plsc.VectorSubcoreMesh(core_axis_name="c", subcore_axis_name="s")


@functools.partial(
    pl.kernel,
    mesh=_mesh,
    out_type=jax.ShapeDtypeStruct((BATCH,), jnp.float32),
    scratch_types=[
        pltpu.VMEM((_ROWS,), jnp.float32),
        pltpu.SemaphoreType.DMA,
    ],
)
def _fifo_write_eps(eps_hbm, out_eps_hbm, ebuf, esem):
    wid = lax.axis_index("s") * _NC + lax.axis_index("c")
    base = wid * _ROWS
    pltpu.async_copy(eps_hbm.at[pl.ds(base, _ROWS)], ebuf, esem).wait()
    pltpu.async_copy(ebuf, out_eps_hbm.at[pl.ds(base, _ROWS)], esem).wait()


_TC_BLOCK = 512


def _ctx_copy_body(cxt_ref, out_ref):
    out_ref[...] = cxt_ref[...]


_ctx_copy = pl.pallas_call(
    _ctx_copy_body,
    grid=(BATCH // _TC_BLOCK,),
    in_specs=[pl.BlockSpec((_TC_BLOCK, CTX_SIZE), lambda i: (i, 0))],
    out_specs=pl.BlockSpec((_TC_BLOCK, CTX_SIZE), lambda i: (i, 0)),
    out_shape=jax.ShapeDtypeStruct((BATCH, CTX_SIZE), jnp.float32),
)


def kernel(mem_ctx, mem_eps, cxt, eps):
    out_eps = _fifo_write_eps(eps.reshape(BATCH))
    out_ctx = _ctx_copy(cxt)
    return out_ctx, out_eps.reshape(BATCH, 1)
